# Initial kernel scaffold; baseline (speedup 1.0000x reference)
import jax
import jax.numpy as jnp
from jax.experimental import pallas as pl


def kernel(x):
    raise NotImplementedError("write your pallas kernel here")


# speedup vs baseline: 4.7986x; 4.7986x over previous
import numpy as np
import jax
import jax.numpy as jnp
from jax.experimental import pallas as pl
from jax.experimental.pallas import tpu as pltpu

_K = 64
_B = 128
_N = 32768
_RB = 8
_IMIN = np.int32(-2147483648)


def _topk_mask_body(x_ref, o_ref):
    x = x_ref[...]
    bits = jax.lax.bitcast_convert_type(x, jnp.int32)
    v = jnp.where(bits >= 0, bits, bits ^ jnp.int32(0x7FFFFFFF))

    def count_ge(cand_s):
        return jnp.sum((v >= cand_s).astype(jnp.int32), axis=1, keepdims=True)

    def step(i, t_u):
        bit = jnp.left_shift(jnp.int32(1), jnp.int32(31) - i)
        cand_u = t_u | bit
        cnt = count_ge(cand_u ^ _IMIN)
        return jnp.where(cnt >= _K, cand_u, t_u)

    t_u = jax.lax.fori_loop(
        0, 32, step, jnp.zeros((_RB, 1), jnp.int32), unroll=True
    )
    t_s = t_u ^ _IMIN

    gt = v > t_s
    eq = v == t_s
    cnt_gt = jnp.sum(gt.astype(jnp.int32), axis=1, keepdims=True)
    cnt_eq = jnp.sum(eq.astype(jnp.int32), axis=1, keepdims=True)
    r = _K - cnt_gt

    iota = jax.lax.broadcasted_iota(jnp.int32, (_RB, _N), 1)
    eqi = jnp.where(eq, iota, jnp.int32(_N))

    def tie_step(i, c_t):
        bit = jnp.left_shift(jnp.int32(1), jnp.int32(14) - i)
        cand = c_t | bit
        g = jnp.sum((eqi < cand).astype(jnp.int32), axis=1, keepdims=True)
        return jnp.where(g < r, cand, c_t)

    def tie_search(_):
        return jax.lax.fori_loop(
            0, 15, tie_step, jnp.zeros((_RB, 1), jnp.int32), unroll=True
        )

    need_tie = jnp.any(cnt_gt + cnt_eq > _K)
    c_t = jax.lax.cond(
        need_tie, tie_search, lambda _: jnp.full((_RB, 1), jnp.int32(_N - 1)), None
    )

    keep = gt | (eqi <= c_t)
    o_ref[...] = jnp.where(keep, x, jnp.float32(0.0))


def kernel(x):
    return pl.pallas_call(
        _topk_mask_body,
        grid=(_B // _RB,),
        in_specs=[pl.BlockSpec((_RB, _N), lambda i: (i, 0))],
        out_specs=pl.BlockSpec((_RB, _N), lambda i: (i, 0)),
        out_shape=jax.ShapeDtypeStruct((_B, _N), jnp.float32),
        compiler_params=pltpu.CompilerParams(
            dimension_semantics=("arbitrary",),
        ),
    )(x)


# speedup vs baseline: 6.7940x; 1.4158x over previous
import numpy as np
import jax
import jax.numpy as jnp
from jax.experimental import pallas as pl
from jax.experimental.pallas import tpu as pltpu

_K = 64
_B = 128
_N = 32768
_RB = 8
_IMIN = np.int32(-2147483648)


def _topk_mask_body(x_ref, o_ref):
    x = x_ref[...]
    bits = jax.lax.bitcast_convert_type(x, jnp.int32)
    v = jnp.where(bits >= 0, bits, bits ^ jnp.int32(0x7FFFFFFF))

    sh = (v >> 16).astype(jnp.int16)

    def lane_count(m16):
        w = _N
        while w > 256:
            h = w // 2
            m16 = m16[:, :h] + m16[:, h:w]
            w = h
        return jnp.sum(m16.astype(jnp.int32), axis=1, keepdims=True)

    def as_s16(pat_u):
        return (((pat_u ^ jnp.int32(0x8000)) << 16) >> 16).astype(jnp.int16)

    def step_hi(i, t_u):
        bit = jnp.left_shift(jnp.int32(1), jnp.int32(15) - i)
        cand_u = t_u | bit
        cnt = lane_count((sh >= as_s16(cand_u)).astype(jnp.int16))
        return jnp.where(cnt >= _K, cand_u, t_u)

    th_u = jax.lax.fori_loop(
        0, 16, step_hi, jnp.zeros((_RB, 1), jnp.int32), unroll=True
    )
    th_s16 = as_s16(th_u)

    cnt_gt_hi = lane_count((sh > th_s16).astype(jnp.int16))
    r2 = _K - cnt_gt_hi

    sl = (((v ^ jnp.int32(0x8000)) << 16) >> 16).astype(jnp.int16)
    wl = jnp.where(sh == th_s16, sl, jnp.int16(-32768))

    def step_lo(i, t_u):
        bit = jnp.left_shift(jnp.int32(1), jnp.int32(15) - i)
        cand_u = t_u | bit
        cnt = lane_count((wl >= as_s16(cand_u)).astype(jnp.int16))
        return jnp.where(cnt >= r2, cand_u, t_u)

    tl_u = jax.lax.fori_loop(
        0, 16, step_lo, jnp.zeros((_RB, 1), jnp.int32), unroll=True
    )

    th_sval = ((th_u ^ jnp.int32(0x8000)) << 16) >> 16
    t_s = (th_sval << 16) | tl_u

    gt = v > t_s
    eq = v == t_s
    cnt_gt = jnp.sum(gt.astype(jnp.int32), axis=1, keepdims=True)
    cnt_eq = jnp.sum(eq.astype(jnp.int32), axis=1, keepdims=True)
    r = _K - cnt_gt

    iota = jax.lax.broadcasted_iota(jnp.int32, (_RB, _N), 1)
    eqi = jnp.where(eq, iota, jnp.int32(_N))

    def tie_step(i, c_t):
        bit = jnp.left_shift(jnp.int32(1), jnp.int32(14) - i)
        cand = c_t | bit
        g = jnp.sum((eqi < cand).astype(jnp.int32), axis=1, keepdims=True)
        return jnp.where(g < r, cand, c_t)

    def tie_search(_):
        return jax.lax.fori_loop(
            0, 15, tie_step, jnp.zeros((_RB, 1), jnp.int32), unroll=True
        )

    need_tie = jnp.any(cnt_gt + cnt_eq > _K)
    c_t = jax.lax.cond(
        need_tie, tie_search, lambda _: jnp.full((_RB, 1), jnp.int32(_N - 1)), None
    )

    keep = gt | (eqi <= c_t)
    o_ref[...] = jnp.where(keep, x, jnp.float32(0.0))


def kernel(x):
    return pl.pallas_call(
        _topk_mask_body,
        grid=(_B // _RB,),
        in_specs=[pl.BlockSpec((_RB, _N), lambda i: (i, 0))],
        out_specs=pl.BlockSpec((_RB, _N), lambda i: (i, 0)),
        out_shape=jax.ShapeDtypeStruct((_B, _N), jnp.float32),
        compiler_params=pltpu.CompilerParams(
            dimension_semantics=("arbitrary",),
        ),
    )(x)


# speedup vs baseline: 6.8142x; 1.0030x over previous
import numpy as np
import jax
import jax.numpy as jnp
from jax.experimental import pallas as pl
from jax.experimental.pallas import tpu as pltpu

_K = 64
_B = 128
_N = 32768
_RB = 8
_CW = 2048
_NACC = 4


def _count16(masks16):
    accs = [jnp.zeros((_RB, _CW), jnp.int16) for _ in range(_NACC)]
    for ci, m in enumerate(masks16):
        accs[ci % _NACC] = accs[ci % _NACC] + m
    acc = (accs[0] + accs[1]) + (accs[2] + accs[3])
    w = _CW
    while w > 256:
        h = w // 2
        acc = acc[:, :h] + acc[:, h:w]
        w = h
    return jnp.sum(acc.astype(jnp.int32), axis=1, keepdims=True)


def _count_ge16(data, cand16):
    return _count16(
        (data[:, c : c + _CW] >= cand16).astype(jnp.int16)
        for c in range(0, _N, _CW)
    )


def _count_mask16(m16):
    return _count16(m16[:, c : c + _CW] for c in range(0, _N, _CW))


def _as_s16(pat_u):
    return (((pat_u ^ jnp.int32(0x8000)) << 16) >> 16).astype(jnp.int16)


def _topk_mask_body(x_ref, o_ref):
    x = x_ref[...]
    bits = jax.lax.bitcast_convert_type(x, jnp.int32)
    v = jnp.where(bits >= 0, bits, bits ^ jnp.int32(0x7FFFFFFF))
    sh = (v >> 16).astype(jnp.int16)
    sl = (((v ^ jnp.int32(0x8000)) << 16) >> 16).astype(jnp.int16)

    def step_hi(i, t_u):
        bit = jnp.left_shift(jnp.int32(1), jnp.int32(15) - i)
        cand_u = t_u | bit
        cnt = _count_ge16(sh, _as_s16(cand_u))
        return jnp.where(cnt >= _K, cand_u, t_u)

    th_u = jax.lax.fori_loop(
        0, 16, step_hi, jnp.zeros((_RB, 1), jnp.int32), unroll=True
    )
    th16 = _as_s16(th_u)

    sh_eq = sh == th16
    cnt_gt_hi = _count_mask16((sh > th16).astype(jnp.int16))
    r2 = _K - cnt_gt_hi

    wl = jnp.where(sh_eq, sl, jnp.int16(-32768))

    def step_lo(i, t_u):
        bit = jnp.left_shift(jnp.int32(1), jnp.int32(15) - i)
        cand_u = t_u | bit
        cnt = _count_ge16(wl, _as_s16(cand_u))
        return jnp.where(cnt >= r2, cand_u, t_u)

    tl_u = jax.lax.fori_loop(
        0, 16, step_lo, jnp.zeros((_RB, 1), jnp.int32), unroll=True
    )
    tl16 = _as_s16(tl_u)

    gt = (sh > th16) | (sh_eq & (sl > tl16))
    eq = sh_eq & (sl == tl16)
    cnt_gt = _count_mask16(gt.astype(jnp.int16))
    cnt_eq = _count_mask16(eq.astype(jnp.int16))
    r = _K - cnt_gt

    iota16 = jax.lax.broadcasted_iota(jnp.int32, (_RB, _N), 1).astype(jnp.int16)

    def tie_step(i, c_t):
        bit = jnp.left_shift(jnp.int32(1), jnp.int32(14) - i)
        cand = c_t | bit
        cand16 = cand.astype(jnp.int16)
        g = _count_mask16((eq & (iota16 < cand16)).astype(jnp.int16))
        return jnp.where(g < r, cand, c_t)

    def tie_search(_):
        return jax.lax.fori_loop(
            0, 15, tie_step, jnp.zeros((_RB, 1), jnp.int32), unroll=True
        )

    need_tie = jnp.any(cnt_gt + cnt_eq > _K)
    c_t = jax.lax.cond(
        need_tie, tie_search, lambda _: jnp.full((_RB, 1), jnp.int32(_N - 1)), None
    )
    c_t16 = c_t.astype(jnp.int16)

    keep = gt | (eq & (iota16 <= c_t16))
    o_ref[...] = jnp.where(keep, x, jnp.float32(0.0))


def kernel(x):
    return pl.pallas_call(
        _topk_mask_body,
        grid=(_B // _RB,),
        in_specs=[pl.BlockSpec((_RB, _N), lambda i: (i, 0))],
        out_specs=pl.BlockSpec((_RB, _N), lambda i: (i, 0)),
        out_shape=jax.ShapeDtypeStruct((_B, _N), jnp.float32),
        compiler_params=pltpu.CompilerParams(
            dimension_semantics=("arbitrary",),
        ),
    )(x)
